# baseline (speedup 1.0000x reference)
import math

import jax
import jax.numpy as jnp
from jax.experimental import pallas as pl
from jax.experimental.pallas import tpu as pltpu

NUM_LAYERS = 3
HEADS = 8
C = 64
NJ = 25
CLASSES = 60
B_SEG = 256
F_FRAMES = 8192
E_RAW = 50
E_PAD = 64
DH = C // HEADS
EPS = 1e-5

FB = 64
NBLK = F_FRAMES // FB

_INTERPRET = False


def _layer_prologue(y, a, st_ref):
    scale = 1.0 / (F_FRAMES * C)
    s0 = st_ref[0:1, 0:NJ]
    s1 = st_ref[1:2, 0:NJ]
    mu = s0 * scale
    var = s1 * scale - mu * mu
    inv = jax.lax.rsqrt(var + EPS)
    mu3 = mu.reshape(1, NJ, 1)
    inv3 = inv.reshape(1, NJ, 1)
    return jnp.maximum((y - mu3) * inv3 + a, 0.0)


def _hga_encoder(z, adj, W, asrc, adst, Wq, Wk, Wv, Wo1, W2):
    iota_ne = jax.lax.broadcasted_iota(jnp.int32, (NJ, E_PAD), 0)
    St = (adj[0:1, :] == iota_ne).astype(jnp.float32)
    Dt = (adj[1:2, :] == iota_ne).astype(jnp.float32)

    h2 = z.reshape(FB * NJ, C) @ W
    h = h2.reshape(FB, NJ, C)

    p_src = (h2 * asrc).reshape(FB, NJ, HEADS, DH).sum(-1)
    p_dst = (h2 * adst).reshape(FB, NJ, HEADS, DH).sum(-1)
    lo = (jnp.einsum('fnh,ne->feh', p_src, St)
          + jnp.einsum('fnh,ne->feh', p_dst, Dt))
    lo = jnp.where(lo >= 0, lo, 0.2 * lo)
    ew = jnp.exp(lo)

    denom = jnp.einsum('feh,ne->fnh', ew, Dt)
    denom = jnp.where(denom > 0, denom, 1.0)

    h_src = jnp.einsum('fnc,ne->fec', h, St)
    ew_c = jnp.broadcast_to(ew[:, :, :, None],
                            (FB, E_PAD, HEADS, DH)).reshape(FB, E_PAD, C)
    num = jnp.einsum('fec,ne->fnc', h_src * ew_c, Dt)
    den_c = jnp.broadcast_to(denom[:, :, :, None],
                             (FB, NJ, HEADS, DH)).reshape(FB, NJ, C)
    a_out = jnp.maximum(num / den_c, 0.0)

    a2 = a_out.reshape(FB * NJ, C)
    q = (a2 @ Wq).reshape(FB, NJ, C)
    k = (a2 @ Wk).reshape(FB, NJ, C)
    v = (a2 @ Wv).reshape(FB, NJ, C)
    inv_sqrt = 1.0 / math.sqrt(DH)
    o_heads = []
    for hh in range(HEADS):
        sl = slice(hh * DH, (hh + 1) * DH)
        qh = q[:, :, sl]
        kh = k[:, :, sl]
        vh = v[:, :, sl]
        sc = jax.lax.dot_general(
            qh, kh, (((2,), (2,)), ((0,), (0,)))) * inv_sqrt
        aw = jax.nn.softmax(sc, axis=-1)
        o_heads.append(jax.lax.dot_general(
            aw, vh, (((2,), (1,)), ((0,), (0,)))))
    o = jnp.concatenate(o_heads, axis=-1).reshape(FB * NJ, C)
    y = (jnp.maximum(o @ Wo1, 0.0) @ W2).reshape(FB, NJ, C)
    return a_out, y


def _emit_layer_outputs(a_out, y, aout_ref, yout_ref, stout_ref):
    aout_ref[...] = a_out.reshape(FB * NJ, C)
    yout_ref[...] = y.reshape(FB * NJ, C)
    i = pl.program_id(0)

    @pl.when(i == 0)
    def _():
        stout_ref[...] = jnp.zeros((8, 128), jnp.float32)

    ysum = jnp.sum(jnp.sum(y, axis=2), axis=0, keepdims=True)
    ysq = jnp.sum(jnp.sum(y * y, axis=2), axis=0, keepdims=True)
    stout_ref[0:1, 0:NJ] = stout_ref[0:1, 0:NJ] + ysum
    stout_ref[1:2, 0:NJ] = stout_ref[1:2, 0:NJ] + ysq


def _layer0_body(t_ref, adj_ref, W_ref, asrc_ref, adst_ref, Wq_ref, Wk_ref,
                 Wv_ref, Wo1_ref, W2_ref, aout_ref, yout_ref, stout_ref):
    z = t_ref[...].reshape(FB, NJ, C)
    a_out, y = _hga_encoder(z, adj_ref[...], W_ref[...], asrc_ref[...],
                            adst_ref[...], Wq_ref[...], Wk_ref[...],
                            Wv_ref[...], Wo1_ref[...], W2_ref[...])
    _emit_layer_outputs(a_out, y, aout_ref, yout_ref, stout_ref)


def _layer_body(y_ref, a_ref, st_ref, adj_ref, W_ref, asrc_ref, adst_ref,
                Wq_ref, Wk_ref, Wv_ref, Wo1_ref, W2_ref,
                aout_ref, yout_ref, stout_ref):
    y_prev = y_ref[...].reshape(FB, NJ, C)
    a_prev = a_ref[...].reshape(FB, NJ, C)
    z = _layer_prologue(y_prev, a_prev, st_ref)
    a_out, y = _hga_encoder(z, adj_ref[...], W_ref[...], asrc_ref[...],
                            adst_ref[...], Wq_ref[...], Wk_ref[...],
                            Wv_ref[...], Wo1_ref[...], W2_ref[...])
    _emit_layer_outputs(a_out, y, aout_ref, yout_ref, stout_ref)


def _pool_body(y_ref, a_ref, st_ref, bi_ref, Wg_ref, u_ref,
               den_ref, num_ref):
    y_prev = y_ref[...].reshape(FB, NJ, C)
    a_prev = a_ref[...].reshape(FB, NJ, C)
    z = _layer_prologue(y_prev, a_prev, st_ref)

    g = jnp.tanh(z.reshape(FB * NJ, C) @ Wg_ref[...])
    s = jnp.sum(g * u_ref[...], axis=1, keepdims=True).reshape(FB, NJ)
    e = jnp.exp(s)

    bi_col = bi_ref[0]
    onehot = (bi_col == jax.lax.broadcasted_iota(
        jnp.int32, (FB, B_SEG), 1)).astype(jnp.float32)

    den_blk = jax.lax.dot_general(
        onehot, e, (((0,), (0,)), ((), ())))
    w = (z * e[:, :, None]).reshape(FB, NJ * C)
    num_blk = jax.lax.dot_general(
        onehot, w, (((0,), (0,)), ((), ())))

    i = pl.program_id(0)

    @pl.when(i == 0)
    def _():
        den_ref[...] = jnp.zeros((B_SEG, 128), jnp.float32)
        num_ref[...] = jnp.zeros((B_SEG, NJ * C), jnp.float32)

    den_ref[:, 0:NJ] = den_ref[:, 0:NJ] + den_blk
    num_ref[...] = num_ref[...] + num_blk


def _final_body(num_ref, den_ref, Wf_ref, bf_ref, out_ref):
    den = den_ref[:, 0:NJ]
    den = jnp.where(den > 0, den, 1.0)
    inv = 1.0 / den
    inv_c = jnp.broadcast_to(inv[:, :, None],
                             (B_SEG, NJ, C)).reshape(B_SEG, NJ * C)
    pooled = jnp.maximum(num_ref[...] * inv_c, 0.0)
    out_ref[...] = pooled @ Wf_ref[...] + bf_ref[...]


def _x_spec():
    return pl.BlockSpec((FB * NJ, C), lambda i: (i, 0))


def _full(shape):
    return pl.BlockSpec(shape, lambda i: (0,) * len(shape))


def _cparams():
    return pltpu.CompilerParams(dimension_semantics=("arbitrary",))


def kernel(t, adj, bi, params):
    F_, N, Cc = t.shape
    t2 = t.reshape(F_ * N, Cc)
    adj_p = jnp.full((8, E_PAD), NJ, jnp.int32).at[0:2, 0:E_RAW].set(adj)
    bi3 = bi.reshape(NBLK, FB, 1)

    x_out = jax.ShapeDtypeStruct((F_FRAMES * NJ, C), jnp.float32)
    st_out = jax.ShapeDtypeStruct((8, 128), jnp.float32)

    def layer_weights(p):
        return (p['W'], p['a_src'].reshape(1, C), p['a_dst'].reshape(1, C),
                p['Wq'], p['Wk'], p['Wv'], p['Wo'] @ p['W1'], p['W2'])

    w_specs = [_full((C, C)), _full((1, C)), _full((1, C)), _full((C, C)),
               _full((C, C)), _full((C, C)), _full((C, C)), _full((C, C))]

    p0 = params['layer0']
    a_prev, y_prev, st_prev = pl.pallas_call(
        _layer0_body,
        grid=(NBLK,),
        in_specs=[_x_spec(), _full((8, E_PAD))] + w_specs,
        out_specs=(_x_spec(), _x_spec(), _full((8, 128))),
        out_shape=(x_out, x_out, st_out),
        compiler_params=_cparams(),
        interpret=_INTERPRET,
    )(t2, adj_p, *layer_weights(p0))

    for li in range(1, NUM_LAYERS):
        p = params['layer%d' % li]
        a_prev, y_prev, st_prev = pl.pallas_call(
            _layer_body,
            grid=(NBLK,),
            in_specs=[_x_spec(), _x_spec(), _full((8, 128)),
                      _full((8, E_PAD))] + w_specs,
            out_specs=(_x_spec(), _x_spec(), _full((8, 128))),
            out_shape=(x_out, x_out, st_out),
            compiler_params=_cparams(),
            interpret=_INTERPRET,
        )(y_prev, a_prev, st_prev, adj_p, *layer_weights(p))

    den, num = pl.pallas_call(
        _pool_body,
        grid=(NBLK,),
        in_specs=[_x_spec(), _x_spec(), _full((8, 128)),
                  pl.BlockSpec((1, FB, 1), lambda i: (i, 0, 0)),
                  _full((C, C)), _full((1, C))],
        out_specs=(_full((B_SEG, 128)), _full((B_SEG, NJ * C))),
        out_shape=(jax.ShapeDtypeStruct((B_SEG, 128), jnp.float32),
                   jax.ShapeDtypeStruct((B_SEG, NJ * C), jnp.float32)),
        compiler_params=_cparams(),
        interpret=_INTERPRET,
    )(y_prev, a_prev, st_prev, bi3, params['Wg'], params['u'].reshape(1, C))

    out = pl.pallas_call(
        _final_body,
        grid=(1,),
        in_specs=[_full((B_SEG, NJ * C)), _full((B_SEG, 128)),
                  _full((NJ * C, CLASSES)), _full((1, CLASSES))],
        out_specs=_full((B_SEG, CLASSES)),
        out_shape=jax.ShapeDtypeStruct((B_SEG, CLASSES), jnp.float32),
        compiler_params=_cparams(),
        interpret=_INTERPRET,
    )(num, den, params['Wf'], params['bf'].reshape(1, CLASSES))

    return out


# speedup vs baseline: 6.1007x; 6.1007x over previous
import math

import jax
import jax.numpy as jnp
from jax.experimental import pallas as pl
from jax.experimental.pallas import tpu as pltpu

NUM_LAYERS = 3
HEADS = 8
C = 64
NJ = 25
NP = 32
CLASSES = 60
B_SEG = 256
F_FRAMES = 8192
E_RAW = 50
E_PAD = 64
DH = C // HEADS
EPS = 1e-5

FB = 128
NBLK = F_FRAMES // FB

_INTERPRET = False
_NEG = -1e30


def _stats_total(st_ref):
    tot = jnp.sum(st_ref[...], axis=0)
    scale = 1.0 / (F_FRAMES * C)
    s0 = tot[:, 0:NP]
    s1 = tot[:, NP:2 * NP]
    mu = s0 * scale
    var = s1 * scale - mu * mu
    inv = jax.lax.rsqrt(var + EPS)
    return mu.reshape(1, NP, 1), inv.reshape(1, NP, 1)


def _hga_encoder(z3, adj, W, WAs, WAd, Wq, Wk, Wv, Wo1, W2):
    z2 = z3.reshape(FB * NP, C)
    iota_ne = jax.lax.broadcasted_iota(jnp.int32, (NP, E_PAD), 0)
    St = (adj[0:1, :] == iota_ne).astype(jnp.float32)
    Dt = (adj[1:2, :] == iota_ne).astype(jnp.float32)

    h3 = (z2 @ W).reshape(FB, NP, C)
    p_src = (z2 @ WAs).reshape(FB, NP, HEADS)
    p_dst = (z2 @ WAd).reshape(FB, NP, HEADS)

    lo = (jax.lax.dot_general(p_src, St, (((1,), (0,)), ((), ())))
          + jax.lax.dot_general(p_dst, Dt, (((1,), (0,)), ((), ()))))
    lo = jnp.where(lo >= 0, lo, 0.2 * lo)
    ew = jnp.exp(lo)

    den = jax.lax.dot_general(ew, Dt, (((2,), (1,)), ((), ())))
    den = jnp.where(den > 0, den, 1.0)

    h_src = jax.lax.dot_general(h3, St, (((1,), (0,)), ((), ())))
    ew_c = jnp.broadcast_to(ew[:, :, None, :],
                            (FB, HEADS, DH, E_PAD)).reshape(FB, C, E_PAD)
    msg = jax.lax.dot_general(h_src * ew_c, Dt,
                              (((2,), (1,)), ((), ())))
    den_c = jnp.broadcast_to(den[:, :, None, :],
                             (FB, HEADS, DH, NP)).reshape(FB, C, NP)
    a3 = jnp.swapaxes(jnp.maximum(msg / den_c, 0.0), 1, 2)

    a2 = a3.reshape(FB * NP, C)
    q3 = (a2 @ Wq).reshape(FB, NP, C)
    k3 = (a2 @ Wk).reshape(FB, NP, C)
    v3 = (a2 @ Wv).reshape(FB, NP, C)
    inv_sqrt = 1.0 / math.sqrt(DH)
    head_of = jax.lax.broadcasted_iota(jnp.int32, (1, 1, C), 2) // DH
    m_valid = jax.lax.broadcasted_iota(jnp.int32, (FB, NP, NP), 2) < NJ
    o3 = jnp.zeros((FB, NP, C), jnp.float32)
    for hh in range(HEADS):
        lane_m = head_of == hh
        qm = jnp.where(lane_m, q3, 0.0)
        sc = jax.lax.dot_general(
            qm, k3, (((2,), (2,)), ((0,), (0,)))) * inv_sqrt
        aw = jax.nn.softmax(jnp.where(m_valid, sc, _NEG), axis=-1)
        vm = jnp.where(lane_m, v3, 0.0)
        o3 = o3 + jax.lax.dot_general(aw, vm, (((2,), (1,)), ((0,), (0,))))
    o2 = o3.reshape(FB * NP, C)
    y3 = (jnp.maximum(o2 @ Wo1, 0.0) @ W2).reshape(FB, NP, C)
    n_valid = jax.lax.broadcasted_iota(jnp.int32, (1, NP, 1), 1) < NJ
    y3 = jnp.where(n_valid, y3, 0.0)
    return a3, y3


def _emit(a3, y3, aout_ref, yout_ref, stout_ref):
    aout_ref[...] = a3.reshape(FB * NP, C)
    yout_ref[...] = y3.reshape(FB * NP, C)
    ysum = jnp.sum(jnp.sum(y3, axis=2), axis=0, keepdims=True)
    ysq = jnp.sum(jnp.sum(y3 * y3, axis=2), axis=0, keepdims=True)
    stout_ref[...] = jnp.concatenate([ysum, ysq], axis=1).reshape(1, 1, 2 * NP)


def _layer0_body(t_ref, adj_ref, W_ref, WAs_ref, WAd_ref, Wq_ref, Wk_ref,
                 Wv_ref, Wo1_ref, W2_ref, aout_ref, yout_ref, stout_ref):
    z3 = t_ref[...].reshape(FB, NP, C)
    a3, y3 = _hga_encoder(z3, adj_ref[...], W_ref[...], WAs_ref[...],
                          WAd_ref[...], Wq_ref[...], Wk_ref[...],
                          Wv_ref[...], Wo1_ref[...], W2_ref[...])
    _emit(a3, y3, aout_ref, yout_ref, stout_ref)


def _layer_body(y_ref, a_ref, st_ref, adj_ref, W_ref, WAs_ref, WAd_ref,
                Wq_ref, Wk_ref, Wv_ref, Wo1_ref, W2_ref,
                aout_ref, yout_ref, stout_ref):
    mu3, inv3 = _stats_total(st_ref)
    y_prev = y_ref[...].reshape(FB, NP, C)
    a_prev = a_ref[...].reshape(FB, NP, C)
    z3 = jnp.maximum((y_prev - mu3) * inv3 + a_prev, 0.0)
    a3, y3 = _hga_encoder(z3, adj_ref[...], W_ref[...], WAs_ref[...],
                          WAd_ref[...], Wq_ref[...], Wk_ref[...],
                          Wv_ref[...], Wo1_ref[...], W2_ref[...])
    _emit(a3, y3, aout_ref, yout_ref, stout_ref)


def _pool_body(y_ref, a_ref, st_ref, bi_ref, Wg_ref, u_ref,
               den_ref, num_ref):
    mu3, inv3 = _stats_total(st_ref)
    y_prev = y_ref[...].reshape(FB, NP, C)
    a_prev = a_ref[...].reshape(FB, NP, C)
    z3 = jnp.maximum((y_prev - mu3) * inv3 + a_prev, 0.0)

    g3 = jnp.tanh((z3.reshape(FB * NP, C) @ Wg_ref[...])).reshape(FB, NP, C)
    s = jnp.sum(g3 * u_ref[...].reshape(1, 1, C), axis=2)
    e = jnp.exp(s)

    bi_col = bi_ref[0]
    onehot = (bi_col == jax.lax.broadcasted_iota(
        jnp.int32, (FB, B_SEG), 1)).astype(jnp.float32)

    den_blk = jax.lax.dot_general(onehot, e, (((0,), (0,)), ((), ())))
    w2 = (z3 * e[:, :, None]).reshape(FB, NP * C)
    num_blk = jax.lax.dot_general(onehot, w2, (((0,), (0,)), ((), ())))

    i = pl.program_id(0)

    @pl.when(i == 0)
    def _():
        den_ref[...] = jnp.zeros((B_SEG, NP), jnp.float32)
        num_ref[...] = jnp.zeros((B_SEG, NP * C), jnp.float32)

    den_ref[...] = den_ref[...] + den_blk
    num_ref[...] = num_ref[...] + num_blk


def _final_body(num_ref, den_ref, Wf_ref, bf_ref, out_ref):
    den = den_ref[...]
    den = jnp.where(den > 0, den, 1.0)
    inv_c = jnp.broadcast_to((1.0 / den)[:, :, None],
                             (B_SEG, NP, C)).reshape(B_SEG, NP * C)
    pooled = jnp.maximum(num_ref[...] * inv_c, 0.0)
    out_ref[...] = pooled @ Wf_ref[...] + bf_ref[...]


def _x_spec():
    return pl.BlockSpec((FB * NP, C), lambda i: (i, 0))


def _full(shape):
    return pl.BlockSpec(shape, lambda i: (0,) * len(shape))


def kernel(t, adj, bi, params):
    F_, N, Cc = t.shape
    t32 = jnp.pad(t, ((0, 0), (0, NP - N), (0, 0))).reshape(F_ * NP, Cc)
    adj_p = jnp.full((8, E_PAD), 127, jnp.int32).at[0:2, 0:E_RAW].set(adj)
    bi3 = bi.reshape(NBLK, FB, 1)

    x_out = jax.ShapeDtypeStruct((F_FRAMES * NP, C), jnp.float32)
    st_out = jax.ShapeDtypeStruct((NBLK, 1, 2 * NP), jnp.float32)
    st_spec = pl.BlockSpec((1, 1, 2 * NP), lambda i: (i, 0, 0))

    head_sel = (jnp.arange(C)[:, None] // DH
                == jnp.arange(HEADS)[None, :]).astype(jnp.float32)

    def layer_weights(p):
        As = p['a_src'].reshape(-1)[:, None] * head_sel
        Ad = p['a_dst'].reshape(-1)[:, None] * head_sel
        return (p['W'], p['W'] @ As, p['W'] @ Ad,
                p['Wq'], p['Wk'], p['Wv'], p['Wo'] @ p['W1'], p['W2'])

    w_specs = [_full((C, C)), _full((C, HEADS)), _full((C, HEADS)),
               _full((C, C)), _full((C, C)), _full((C, C)), _full((C, C)),
               _full((C, C))]

    par = pltpu.CompilerParams(dimension_semantics=("parallel",))
    seq = pltpu.CompilerParams(dimension_semantics=("arbitrary",))

    a_prev, y_prev, st_prev = pl.pallas_call(
        _layer0_body,
        grid=(NBLK,),
        in_specs=[_x_spec(), _full((8, E_PAD))] + w_specs,
        out_specs=(_x_spec(), _x_spec(), st_spec),
        out_shape=(x_out, x_out, st_out),
        compiler_params=par,
        interpret=_INTERPRET,
    )(t32, adj_p, *layer_weights(params['layer0']))

    for li in range(1, NUM_LAYERS):
        a_prev, y_prev, st_prev = pl.pallas_call(
            _layer_body,
            grid=(NBLK,),
            in_specs=[_x_spec(), _x_spec(), _full((NBLK, 1, 2 * NP)),
                      _full((8, E_PAD))] + w_specs,
            out_specs=(_x_spec(), _x_spec(), st_spec),
            out_shape=(x_out, x_out, st_out),
            compiler_params=par,
            interpret=_INTERPRET,
        )(y_prev, a_prev, st_prev, adj_p, *layer_weights(params['layer%d' % li]))

    den, num = pl.pallas_call(
        _pool_body,
        grid=(NBLK,),
        in_specs=[_x_spec(), _x_spec(), _full((NBLK, 1, 2 * NP)),
                  pl.BlockSpec((1, FB, 1), lambda i: (i, 0, 0)),
                  _full((C, C)), _full((1, C))],
        out_specs=(_full((B_SEG, NP)), _full((B_SEG, NP * C))),
        out_shape=(jax.ShapeDtypeStruct((B_SEG, NP), jnp.float32),
                   jax.ShapeDtypeStruct((B_SEG, NP * C), jnp.float32)),
        compiler_params=seq,
        interpret=_INTERPRET,
    )(y_prev, a_prev, st_prev, bi3, params['Wg'], params['u'].reshape(1, C))

    Wf_pad = jnp.zeros((NP, C, CLASSES), jnp.float32).at[:NJ].set(
        params['Wf'].reshape(NJ, C, CLASSES)).reshape(NP * C, CLASSES)

    out = pl.pallas_call(
        _final_body,
        grid=(1,),
        in_specs=[_full((B_SEG, NP * C)), _full((B_SEG, NP)),
                  _full((NP * C, CLASSES)), _full((1, CLASSES))],
        out_specs=_full((B_SEG, CLASSES)),
        out_shape=jax.ShapeDtypeStruct((B_SEG, CLASSES), jnp.float32),
        compiler_params=seq,
        interpret=_INTERPRET,
    )(num, den, Wf_pad, params['bf'].reshape(1, CLASSES))

    return out


# speedup vs baseline: 6.4671x; 1.0601x over previous
import math

import jax
import jax.numpy as jnp
from jax.experimental import pallas as pl
from jax.experimental.pallas import tpu as pltpu

NUM_LAYERS = 3
HEADS = 8
C = 64
NJ = 25
NP = 32
CLASSES = 60
B_SEG = 256
F_FRAMES = 8192
E_RAW = 50
E_PAD = 64
DH = C // HEADS
EPS = 1e-5

FB = 256
NBLK = F_FRAMES // FB

_INTERPRET = False
_NEG = -1e30


def _stats_total(st_ref):
    tot = jnp.sum(st_ref[...], axis=0)
    scale = 1.0 / (F_FRAMES * C)
    s0 = tot[:, 0:NP]
    s1 = tot[:, NP:2 * NP]
    mu = s0 * scale
    var = s1 * scale - mu * mu
    inv = jax.lax.rsqrt(var + EPS)
    return mu.reshape(1, NP, 1), inv.reshape(1, NP, 1)


def _hga_encoder(z3, adj, W, WAs, WAd, Wq, Wk, Wv, Wo1, W2):
    z2 = z3.reshape(FB * NP, C)
    iota_ne = jax.lax.broadcasted_iota(jnp.int32, (NP, E_PAD), 0)
    St = (adj[0:1, :] == iota_ne).astype(jnp.float32)
    Dt = (adj[1:2, :] == iota_ne).astype(jnp.float32)

    h3 = (z2 @ W).reshape(FB, NP, C)
    p_src = (z2 @ WAs).reshape(FB, NP, HEADS)
    p_dst = (z2 @ WAd).reshape(FB, NP, HEADS)

    lo = (jax.lax.dot_general(p_src, St, (((1,), (0,)), ((), ())))
          + jax.lax.dot_general(p_dst, Dt, (((1,), (0,)), ((), ()))))
    lo = jnp.where(lo >= 0, lo, 0.2 * lo)
    ew = jnp.exp(lo)

    den = jax.lax.dot_general(ew, Dt, (((2,), (1,)), ((), ())))
    den = jnp.where(den > 0, den, 1.0)

    h_src = jax.lax.dot_general(h3, St, (((1,), (0,)), ((), ())))
    ew_c = jnp.broadcast_to(ew[:, :, None, :],
                            (FB, HEADS, DH, E_PAD)).reshape(FB, C, E_PAD)
    msg = jax.lax.dot_general(h_src * ew_c, Dt,
                              (((2,), (1,)), ((), ())))
    den_c = jnp.broadcast_to(den[:, :, None, :],
                             (FB, HEADS, DH, NP)).reshape(FB, C, NP)
    a3 = jnp.swapaxes(jnp.maximum(msg / den_c, 0.0), 1, 2)

    a2 = a3.reshape(FB * NP, C)
    q3 = (a2 @ Wq).reshape(FB, NP, C)
    k3 = (a2 @ Wk).reshape(FB, NP, C)
    v3 = (a2 @ Wv).reshape(FB, NP, C)
    inv_sqrt = 1.0 / math.sqrt(DH)
    head_of = jax.lax.broadcasted_iota(jnp.int32, (1, 1, C), 2) // DH
    m_valid = jax.lax.broadcasted_iota(jnp.int32, (FB, NP, NP), 2) < NJ
    o3 = jnp.zeros((FB, NP, C), jnp.float32)
    for hh in range(HEADS):
        lane_m = head_of == hh
        qm = jnp.where(lane_m, q3, 0.0)
        sc = jax.lax.dot_general(
            qm, k3, (((2,), (2,)), ((0,), (0,)))) * inv_sqrt
        aw = jax.nn.softmax(jnp.where(m_valid, sc, _NEG), axis=-1)
        vm = jnp.where(lane_m, v3, 0.0)
        o3 = o3 + jax.lax.dot_general(aw, vm, (((2,), (1,)), ((0,), (0,))))
    o2 = o3.reshape(FB * NP, C)
    y3 = (jnp.maximum(o2 @ Wo1, 0.0) @ W2).reshape(FB, NP, C)
    n_valid = jax.lax.broadcasted_iota(jnp.int32, (1, NP, 1), 1) < NJ
    y3 = jnp.where(n_valid, y3, 0.0)
    return a3, y3


def _emit(a3, y3, aout_ref, yout_ref, stout_ref):
    aout_ref[...] = a3.reshape(FB * NP, C)
    yout_ref[...] = y3.reshape(FB * NP, C)
    ysum = jnp.sum(jnp.sum(y3, axis=2), axis=0, keepdims=True)
    ysq = jnp.sum(jnp.sum(y3 * y3, axis=2), axis=0, keepdims=True)
    stout_ref[...] = jnp.concatenate([ysum, ysq], axis=1).reshape(1, 1, 2 * NP)


def _layer0_body(t_ref, adj_ref, W_ref, WAs_ref, WAd_ref, Wq_ref, Wk_ref,
                 Wv_ref, Wo1_ref, W2_ref, aout_ref, yout_ref, stout_ref):
    z3 = t_ref[...].reshape(FB, NP, C)
    a3, y3 = _hga_encoder(z3, adj_ref[...], W_ref[...], WAs_ref[...],
                          WAd_ref[...], Wq_ref[...], Wk_ref[...],
                          Wv_ref[...], Wo1_ref[...], W2_ref[...])
    _emit(a3, y3, aout_ref, yout_ref, stout_ref)


def _layer_body(y_ref, a_ref, st_ref, adj_ref, W_ref, WAs_ref, WAd_ref,
                Wq_ref, Wk_ref, Wv_ref, Wo1_ref, W2_ref,
                aout_ref, yout_ref, stout_ref):
    mu3, inv3 = _stats_total(st_ref)
    y_prev = y_ref[...].reshape(FB, NP, C)
    a_prev = a_ref[...].reshape(FB, NP, C)
    z3 = jnp.maximum((y_prev - mu3) * inv3 + a_prev, 0.0)
    a3, y3 = _hga_encoder(z3, adj_ref[...], W_ref[...], WAs_ref[...],
                          WAd_ref[...], Wq_ref[...], Wk_ref[...],
                          Wv_ref[...], Wo1_ref[...], W2_ref[...])
    _emit(a3, y3, aout_ref, yout_ref, stout_ref)


def _pool_body(y_ref, a_ref, st_ref, bi_ref, Wg_ref, u_ref,
               den_ref, num_ref):
    mu3, inv3 = _stats_total(st_ref)
    y_prev = y_ref[...].reshape(FB, NP, C)
    a_prev = a_ref[...].reshape(FB, NP, C)
    z3 = jnp.maximum((y_prev - mu3) * inv3 + a_prev, 0.0)

    g3 = jnp.tanh((z3.reshape(FB * NP, C) @ Wg_ref[...])).reshape(FB, NP, C)
    s = jnp.sum(g3 * u_ref[...].reshape(1, 1, C), axis=2)
    e = jnp.exp(s)

    bi_col = bi_ref[0]
    onehot = (bi_col == jax.lax.broadcasted_iota(
        jnp.int32, (FB, B_SEG), 1)).astype(jnp.float32)

    den_blk = jax.lax.dot_general(onehot, e, (((0,), (0,)), ((), ())))
    w2 = (z3 * e[:, :, None]).reshape(FB, NP * C)
    num_blk = jax.lax.dot_general(onehot, w2, (((0,), (0,)), ((), ())))

    i = pl.program_id(0)

    @pl.when(i == 0)
    def _():
        den_ref[...] = jnp.zeros((B_SEG, NP), jnp.float32)
        num_ref[...] = jnp.zeros((B_SEG, NP * C), jnp.float32)

    den_ref[...] = den_ref[...] + den_blk
    num_ref[...] = num_ref[...] + num_blk


def _final_body(num_ref, den_ref, Wf_ref, bf_ref, out_ref):
    den = den_ref[...]
    den = jnp.where(den > 0, den, 1.0)
    inv_c = jnp.broadcast_to((1.0 / den)[:, :, None],
                             (B_SEG, NP, C)).reshape(B_SEG, NP * C)
    pooled = jnp.maximum(num_ref[...] * inv_c, 0.0)
    out_ref[...] = pooled @ Wf_ref[...] + bf_ref[...]


def _x_spec():
    return pl.BlockSpec((FB * NP, C), lambda i: (i, 0))


def _full(shape):
    return pl.BlockSpec(shape, lambda i: (0,) * len(shape))


def kernel(t, adj, bi, params):
    F_, N, Cc = t.shape
    t32 = jnp.pad(t, ((0, 0), (0, NP - N), (0, 0))).reshape(F_ * NP, Cc)
    adj_p = jnp.full((8, E_PAD), 127, jnp.int32).at[0:2, 0:E_RAW].set(adj)
    bi3 = bi.reshape(NBLK, FB, 1)

    x_out = jax.ShapeDtypeStruct((F_FRAMES * NP, C), jnp.float32)
    st_out = jax.ShapeDtypeStruct((NBLK, 1, 2 * NP), jnp.float32)
    st_spec = pl.BlockSpec((1, 1, 2 * NP), lambda i: (i, 0, 0))

    head_sel = (jnp.arange(C)[:, None] // DH
                == jnp.arange(HEADS)[None, :]).astype(jnp.float32)

    def layer_weights(p):
        As = p['a_src'].reshape(-1)[:, None] * head_sel
        Ad = p['a_dst'].reshape(-1)[:, None] * head_sel
        return (p['W'], p['W'] @ As, p['W'] @ Ad,
                p['Wq'], p['Wk'], p['Wv'], p['Wo'] @ p['W1'], p['W2'])

    w_specs = [_full((C, C)), _full((C, HEADS)), _full((C, HEADS)),
               _full((C, C)), _full((C, C)), _full((C, C)), _full((C, C)),
               _full((C, C))]

    par = pltpu.CompilerParams(dimension_semantics=("parallel",))
    seq = pltpu.CompilerParams(dimension_semantics=("arbitrary",))

    a_prev, y_prev, st_prev = pl.pallas_call(
        _layer0_body,
        grid=(NBLK,),
        in_specs=[_x_spec(), _full((8, E_PAD))] + w_specs,
        out_specs=(_x_spec(), _x_spec(), st_spec),
        out_shape=(x_out, x_out, st_out),
        compiler_params=par,
        interpret=_INTERPRET,
    )(t32, adj_p, *layer_weights(params['layer0']))

    for li in range(1, NUM_LAYERS):
        a_prev, y_prev, st_prev = pl.pallas_call(
            _layer_body,
            grid=(NBLK,),
            in_specs=[_x_spec(), _x_spec(), _full((NBLK, 1, 2 * NP)),
                      _full((8, E_PAD))] + w_specs,
            out_specs=(_x_spec(), _x_spec(), st_spec),
            out_shape=(x_out, x_out, st_out),
            compiler_params=par,
            interpret=_INTERPRET,
        )(y_prev, a_prev, st_prev, adj_p, *layer_weights(params['layer%d' % li]))

    den, num = pl.pallas_call(
        _pool_body,
        grid=(NBLK,),
        in_specs=[_x_spec(), _x_spec(), _full((NBLK, 1, 2 * NP)),
                  pl.BlockSpec((1, FB, 1), lambda i: (i, 0, 0)),
                  _full((C, C)), _full((1, C))],
        out_specs=(_full((B_SEG, NP)), _full((B_SEG, NP * C))),
        out_shape=(jax.ShapeDtypeStruct((B_SEG, NP), jnp.float32),
                   jax.ShapeDtypeStruct((B_SEG, NP * C), jnp.float32)),
        compiler_params=seq,
        interpret=_INTERPRET,
    )(y_prev, a_prev, st_prev, bi3, params['Wg'], params['u'].reshape(1, C))

    Wf_pad = jnp.zeros((NP, C, CLASSES), jnp.float32).at[:NJ].set(
        params['Wf'].reshape(NJ, C, CLASSES)).reshape(NP * C, CLASSES)

    out = pl.pallas_call(
        _final_body,
        grid=(1,),
        in_specs=[_full((B_SEG, NP * C)), _full((B_SEG, NP)),
                  _full((NP * C, CLASSES)), _full((1, CLASSES))],
        out_specs=_full((B_SEG, CLASSES)),
        out_shape=jax.ShapeDtypeStruct((B_SEG, CLASSES), jnp.float32),
        compiler_params=seq,
        interpret=_INTERPRET,
    )(num, den, Wf_pad, params['bf'].reshape(1, CLASSES))

    return out
